# speedup vs baseline: 6.0835x; 6.0835x over previous
import jax
import jax.numpy as jnp
from jax import lax
from jax.experimental import pallas as pl

_V = 16
_D = 4


def _tc_body(ids_ref, e_ref, h_ref, out_ref):
    t = jnp.dot(e_ref[...], h_ref[...].T, preferred_element_type=jnp.float32)

    pi = lax.broadcasted_iota(jnp.int32, (128, _V), 0)
    ki = lax.broadcasted_iota(jnp.int32, (128, _V), 1)
    a = (pi % _V == ki).astype(jnp.float32)
    tt = jnp.dot(jnp.dot(a, t, preferred_element_type=jnp.float32), a.T,
                 preferred_element_type=jnp.float32)
    bp = lax.broadcasted_iota(jnp.int32, (128, 128), 0) // _V
    bq = lax.broadcasted_iota(jnp.int32, (128, 128), 1) // _V
    w = jnp.where(bp == bq, tt, 0.0)

    si = lax.broadcasted_iota(jnp.int32, (8, 128), 0)
    ji = lax.broadcasted_iota(jnp.int32, (8, 128), 1)
    r = (ji // _V == si).astype(jnp.float32)

    ids_f = ids_ref[...].astype(jnp.float32)
    idse = jnp.dot(ids_f, r, preferred_element_type=jnp.float32)
    kpat = (lax.broadcasted_iota(jnp.int32, idse.shape, 1) % _V).astype(
        jnp.float32)
    oh = (idse == kpat).astype(jnp.float32)
    out_ref[...] = jnp.dot(oh, w, preferred_element_type=jnp.float32)


def kernel(ids, embed_table, head_weight):
    b, l = ids.shape
    n = b * l
    m_tot = n // 8
    mb = 4096
    grid = (m_tot // mb,)

    ids8 = ids.reshape(m_tot, 8)
    out_flat = pl.pallas_call(
        _tc_body,
        grid=grid,
        in_specs=[
            pl.BlockSpec((mb, 8), lambda i: (i, 0)),
            pl.BlockSpec((_V, _D), lambda i: (0, 0)),
            pl.BlockSpec((_V, _D), lambda i: (0, 0)),
        ],
        out_specs=pl.BlockSpec((mb, 128), lambda i: (i, 0)),
        out_shape=jax.ShapeDtypeStruct((m_tot, 128), jnp.float32),
    )(ids8, embed_table, head_weight)
    return out_flat.reshape(b, l, _V)


# speedup vs baseline: 73.9521x; 12.1562x over previous
import jax
import jax.numpy as jnp
from jax import lax
from jax.experimental import pallas as pl

_V = 16
_D = 4
_LB = 8
_BB = 2048


def _tc_body(idst_ref, e_ref, h_ref, out_ref):
    t2 = jnp.dot(h_ref[...], e_ref[...].T, preferred_element_type=jnp.float32)
    kio = lax.broadcasted_iota(jnp.int32, (_V, _BB), 0)
    for l in range(_LB):
        row = idst_ref[l, :].reshape(1, _BB)
        oh = (jnp.broadcast_to(row, (_V, _BB)) == kio).astype(jnp.float32)
        out_ref[l, :, :] = jnp.dot(t2, oh, preferred_element_type=jnp.float32)


def kernel(ids, embed_table, head_weight):
    b, l = ids.shape
    idst = ids.T
    out_t = pl.pallas_call(
        _tc_body,
        grid=(l // _LB, b // _BB),
        in_specs=[
            pl.BlockSpec((_LB, _BB), lambda i, j: (i, j)),
            pl.BlockSpec((_V, _D), lambda i, j: (0, 0)),
            pl.BlockSpec((_V, _D), lambda i, j: (0, 0)),
        ],
        out_specs=pl.BlockSpec((_LB, _V, _BB), lambda i, j: (i, 0, j)),
        out_shape=jax.ShapeDtypeStruct((l, _V, b), jnp.float32),
    )(idst, embed_table, head_weight)
    return out_t.transpose(2, 0, 1)


# speedup vs baseline: 176.4369x; 2.3858x over previous
import jax
import jax.numpy as jnp
from jax import lax
from jax.experimental import pallas as pl

_V = 16
_D = 4
_LB = 8
_BB = 16384


def _tc_body(idst_ref, e_ref, h_ref, out_ref):
    t2 = jnp.dot(h_ref[...], e_ref[...].T, preferred_element_type=jnp.float32)
    kio = lax.broadcasted_iota(jnp.int32, (_V, _BB), 0)
    for l in range(_LB):
        row = idst_ref[l, :].reshape(1, _BB)
        oh = (jnp.broadcast_to(row, (_V, _BB)) == kio).astype(jnp.float32)
        out_ref[l, :, :] = jnp.dot(t2, oh, preferred_element_type=jnp.float32)


def kernel(ids, embed_table, head_weight):
    b, l = ids.shape
    idst = ids.T
    out_t = pl.pallas_call(
        _tc_body,
        grid=(l // _LB, b // _BB),
        in_specs=[
            pl.BlockSpec((_LB, _BB), lambda i, j: (i, j)),
            pl.BlockSpec((_V, _D), lambda i, j: (0, 0)),
            pl.BlockSpec((_V, _D), lambda i, j: (0, 0)),
        ],
        out_specs=pl.BlockSpec((_LB, _V, _BB), lambda i, j: (i, 0, j)),
        out_shape=jax.ShapeDtypeStruct((l, _V, b), jnp.float32),
    )(idst, embed_table, head_weight)
    return out_t.transpose(2, 0, 1)
